# baseline (speedup 1.0000x reference)
import functools

import jax
import jax.numpy as jnp
from jax import lax
from jax.experimental import pallas as pl
from jax.experimental.pallas import tpu as pltpu
from jax.experimental.pallas import tpu_sc as plsc

_N = 50000
_D = 64
_H = 32
_E = 800000
_BATCH = 50000
_DECAY = 1e-05

_NT = 16
_EPT = _E // _NT
_C = 400
_NCH = _EPT // _C
_CW = 16

_mesh = plsc.VectorSubcoreMesh(core_axis_name="c", subcore_axis_name="s")
_sc_params = pltpu.CompilerParams(use_tc_tiling_on_sc=False)
_f32 = jnp.float32
_i32 = jnp.int32


def _fill_const(ref, nrows, width, val):
    def body(r, _):
        for h in range(width // 16):
            ref[r, pl.ds(h * 16, 16)] = jnp.full((16,), val, _f32)
        return 0
    lax.fori_loop(0, nrows, body, 0)


_NRCH = _N // _C
_NJJ = (_NRCH + _NT - 1) // _NT


def _row_chunks(s, fn):
    def body(jj, _):
        ch = s + _NT * jj

        @pl.when(ch < _NRCH)
        def _():
            fn(ch * _C)
        return 0

    lax.fori_loop(0, _NJJ, body, 0)


def _zero_accum(zbuf, accum, s):
    _row_chunks(s, lambda r0: pltpu.sync_copy(zbuf, accum.at[pl.ds(r0, _C)]))


@functools.partial(
    pl.kernel,
    mesh=_mesh,
    out_type=[jax.ShapeDtypeStruct((_N, _CW), _f32),
              jax.ShapeDtypeStruct((_N, _CW), _f32)],
    scratch_types=[
        pltpu.VMEM((_C,), _i32),
        pltpu.VMEM((_C, _CW), _f32),
        pltpu.VMEM((_C, _CW), _f32),
        pltpu.VMEM_SHARED((_N, _CW), _f32),
    ],
    compiler_params=_sc_params,
)
def _sc_counts(src_hbm, dst_hbm, cnt_i_hbm, cnt_u_hbm, idxb, ones, zbuf, accum):
    c = lax.axis_index("c")
    s = lax.axis_index("s")
    _fill_const(ones, _C, _CW, 1.0)
    _fill_const(zbuf, _C, _CW, 0.0)
    _zero_accum(zbuf, accum, s)
    plsc.subcore_barrier()

    def body(j, _):
        e0 = s * _EPT + j * _C

        @pl.when(c == 0)
        def _():
            pltpu.sync_copy(dst_hbm.at[pl.ds(e0, _C)], idxb)

        @pl.when(c == 1)
        def _():
            pltpu.sync_copy(src_hbm.at[pl.ds(e0, _C)], idxb)

        pltpu.sync_copy(ones, accum.at[idxb], add=True)
        return 0

    lax.fori_loop(0, _NCH, body, 0)
    plsc.subcore_barrier()

    @pl.when(c == 0)
    def _():
        _row_chunks(s, lambda r0: pltpu.sync_copy(
            accum.at[pl.ds(r0, _C)], cnt_i_hbm.at[pl.ds(r0, _C)]))

    @pl.when(c == 1)
    def _():
        _row_chunks(s, lambda r0: pltpu.sync_copy(
            accum.at[pl.ds(r0, _C)], cnt_u_hbm.at[pl.ds(r0, _C)]))


@functools.partial(
    pl.kernel,
    mesh=_mesh,
    out_type=[jax.ShapeDtypeStruct((_N, _H), _f32) for _ in range(2)],
    scratch_types=[
        pltpu.VMEM((_C,), _i32),
        pltpu.VMEM((_C,), _i32),
        pltpu.VMEM((_C,), _i32),
        pltpu.VMEM((_C,), _i32),
        pltpu.VMEM((_C, _H), _f32),
        pltpu.VMEM((_C, _H), _f32),
        pltpu.VMEM_SHARED((_N, _H), _f32),
        pltpu.SemaphoreType.DMA,
        pltpu.SemaphoreType.DMA,
    ],
    compiler_params=_sc_params,
)
def _sc_segdir(tbl0, tbl1, g_hbm, s_hbm, out0, out1,
               gidx0, gidx1, sidx0, sidx1, rows0, rows1, accum, sem0, sem1):
    c = lax.axis_index("c")
    s = lax.axis_index("s")
    gidx = (gidx0, gidx1)
    sidx = (sidx0, sidx1)
    rows = (rows0, rows1)
    sems = (sem0, sem1)

    _fill_const(rows0, _C, _H, 0.0)
    _zero_accum(rows0, accum, s)
    plsc.subcore_barrier()

    def issue(b, j):
        e0 = s * _EPT + j * _C
        pltpu.sync_copy(g_hbm.at[pl.ds(e0, _C)], gidx[b])
        pltpu.sync_copy(s_hbm.at[pl.ds(e0, _C)], sidx[b])

        @pl.when(c == 0)
        def _():
            pltpu.async_copy(tbl0.at[gidx[b]], rows[b], sems[b])

        @pl.when(c == 1)
        def _():
            pltpu.async_copy(tbl1.at[gidx[b]], rows[b], sems[b])

    def drain(b):
        pltpu.make_async_copy(tbl0.at[pl.ds(0, _C)], rows[b],
                              sems[b]).wait()
        pltpu.sync_copy(rows[b], accum.at[sidx[b]], add=True)

    issue(0, 0)
    issue(1, 1)

    def body(jj, _):
        for b in range(2):
            drain(b)
            jn = 2 * jj + b + 2

            @pl.when(jn < _NCH)
            def _():
                issue(b, jn)
        return 0

    lax.fori_loop(0, _NCH // 2, body, 0)
    drain(0)
    plsc.subcore_barrier()

    @pl.when(c == 0)
    def _():
        _row_chunks(s, lambda r0: pltpu.sync_copy(
            accum.at[pl.ds(r0, _C)], out0.at[pl.ds(r0, _C)]))

    @pl.when(c == 1)
    def _():
        _row_chunks(s, lambda r0: pltpu.sync_copy(
            accum.at[pl.ds(r0, _C)], out1.at[pl.ds(r0, _C)]))


_GCH = _N // _C
_GJJ = (_GCH + _NT - 1) // _NT


@functools.partial(
    pl.kernel,
    mesh=_mesh,
    out_type=[jax.ShapeDtypeStruct((_N, _H), _f32) for _ in range(4)],
    scratch_types=[
        pltpu.VMEM((_C,), _i32),
        pltpu.VMEM((_C, _H), _f32),
        pltpu.SemaphoreType.DMA,
    ],
    compiler_params=_sc_params,
)
def _sc_gather_pairs(v0_hbm, v1_hbm, pos_hbm, neg_hbm,
                     pv0_hbm, pv1_hbm, nv0_hbm, nv1_hbm, idxb, rows, sem):
    c = lax.axis_index("c")
    s = lax.axis_index("s")

    def body(jj, _):
        ch = s + _NT * jj

        @pl.when(ch < _GCH)
        def _():
            b0 = ch * _C
            pltpu.sync_copy(pos_hbm.at[pl.ds(b0, _C)], idxb)

            @pl.when(c == 0)
            def _():
                pltpu.async_copy(v0_hbm.at[idxb], rows, sem).wait()
                pltpu.sync_copy(rows, pv0_hbm.at[pl.ds(b0, _C)])

            @pl.when(c == 1)
            def _():
                pltpu.async_copy(v1_hbm.at[idxb], rows, sem).wait()
                pltpu.sync_copy(rows, pv1_hbm.at[pl.ds(b0, _C)])

            pltpu.sync_copy(neg_hbm.at[pl.ds(b0, _C)], idxb)

            @pl.when(c == 0)
            def _():
                pltpu.async_copy(v0_hbm.at[idxb], rows, sem).wait()
                pltpu.sync_copy(rows, nv0_hbm.at[pl.ds(b0, _C)])

            @pl.when(c == 1)
            def _():
                pltpu.async_copy(v1_hbm.at[idxb], rows, sem).wait()
                pltpu.sync_copy(rows, nv1_hbm.at[pl.ds(b0, _C)])

        return 0

    lax.fori_loop(0, _GJJ, body, 0)


_R = 2000


def _tc_cat_body(relu, xcat, wcat, out):
    r = jnp.dot(xcat[:, :], wcat[:, :], preferred_element_type=_f32)
    if relu:
        r = jnp.maximum(r, 0.0)
    out[:, :] = r


def _tc_cat(relu):
    return pl.pallas_call(
        functools.partial(_tc_cat_body, relu),
        grid=(_N // _R,),
        in_specs=[pl.BlockSpec((_R, 2 * _D), lambda i: (i, 0)),
                  pl.BlockSpec((2 * _D, _D), lambda i: (0, 0))],
        out_specs=pl.BlockSpec((_R, _D), lambda i: (i, 0)),
        out_shape=jax.ShapeDtypeStruct((_N, _D), _f32),
    )


def _tc_loss_body(ufull, pn, mf, sq):
    i = pl.program_id(0)
    x = ufull[:, :]
    p = pn[:, 0:_D]
    n = pn[:, _D:2 * _D]
    ps = jnp.sum(x * p, axis=1)
    ns = jnp.sum(x * n, axis=1)
    d = ns - ps
    sp = jnp.maximum(d, 0.0) + jnp.log1p(jnp.exp(-jnp.abs(d)))
    blk_mf = jnp.sum(sp)
    blk_sq = jnp.sum(x * x) + jnp.sum(p * p) + jnp.sum(n * n)

    @pl.when(i == 0)
    def _():
        mf[0, 0] = blk_mf
        sq[0, 0] = blk_sq

    @pl.when(i > 0)
    def _():
        mf[0, 0] += blk_mf
        sq[0, 0] += blk_sq


_tc_loss = pl.pallas_call(
    _tc_loss_body,
    grid=(_N // _R,),
    in_specs=[pl.BlockSpec((_R, _D), lambda i: (i, 0)),
              pl.BlockSpec((_R, 2 * _D), lambda i: (i, 0))],
    out_specs=[pl.BlockSpec(memory_space=pltpu.MemorySpace.SMEM)] * 2,
    out_shape=[jax.ShapeDtypeStruct((1, 1), _f32) for _ in range(2)],
)


def kernel(user_feat, item_feat, edge_index, pos_items_ur, neg_items_ur,
           W_self_0, W_neigh_0, W_self_1, W_neigh_1):
    src = edge_index[0].astype(_i32)
    dst = edge_index[1].astype(_i32)
    pos = pos_items_ur.astype(_i32)
    neg = neg_items_ur.astype(_i32)

    u0, u1 = user_feat[:, :_H], user_feat[:, _H:]
    v0, v1 = item_feat[:, :_H], item_feat[:, _H:]

    cnt_i, cnt_u = _sc_counts(src, dst)
    ri = 1.0 / jnp.maximum(cnt_i[:, 0:1], 1.0)
    ru = 1.0 / jnp.maximum(cnt_u[:, 0:1], 1.0)
    wcat0 = jnp.concatenate([W_self_0.T, W_neigh_0.T], axis=0)
    wcat1 = jnp.concatenate([W_self_1.T, W_neigh_1.T], axis=0)

    si0, si1 = _sc_segdir(u0, u1, src, dst)
    su0, su1 = _sc_segdir(v0, v1, dst, src)
    v1f = _tc_cat(True)(
        jnp.concatenate([v0, v1, si0 * ri, si1 * ri], axis=1), wcat0)
    su20, su21 = _sc_segdir(v1f[:, :_H], v1f[:, _H:], dst, src)
    u1f = _tc_cat(True)(
        jnp.concatenate([u0, u1, su0 * ru, su1 * ru], axis=1), wcat0)
    si20, si21 = _sc_segdir(u1f[:, :_H], u1f[:, _H:], src, dst)
    u2f = _tc_cat(False)(
        jnp.concatenate([u1f, su20 * ru, su21 * ru], axis=1), wcat1)
    v2f = _tc_cat(False)(
        jnp.concatenate([v1f, si20 * ri, si21 * ri], axis=1), wcat1)

    pv0, pv1, nv0, nv1 = _sc_gather_pairs(v2f[:, :_H], v2f[:, _H:], pos, neg)
    pn = jnp.concatenate([pv0, pv1, nv0, nv1], axis=1)
    mf, sq = _tc_loss(u2f, pn)

    mf_loss = (mf[0, 0] / _BATCH).astype(_f32)
    emb_loss = (_DECAY * 0.5 * sq[0, 0] / _BATCH).astype(_f32)
    return (mf_loss, emb_loss)


# speedup vs baseline: 1.3767x; 1.3767x over previous
import functools

import jax
import jax.numpy as jnp
from jax import lax
from jax.experimental import pallas as pl
from jax.experimental.pallas import tpu as pltpu
from jax.experimental.pallas import tpu_sc as plsc

_N = 50000
_D = 64
_H = 32
_E = 800000
_BATCH = 50000
_DECAY = 1e-05

_NT = 16
_EPT = _E // _NT
_C = 400
_NCH = _EPT // _C
_CW = 16

_mesh = plsc.VectorSubcoreMesh(core_axis_name="c", subcore_axis_name="s")
_sc_params = pltpu.CompilerParams(use_tc_tiling_on_sc=False)
_f32 = jnp.float32
_i32 = jnp.int32


def _fill_const(ref, nrows, width, val):
    def body(r, _):
        for h in range(width // 16):
            ref[r, pl.ds(h * 16, 16)] = jnp.full((16,), val, _f32)
        return 0
    lax.fori_loop(0, nrows, body, 0)


_NRCH = _N // _C
_NJJ = (_NRCH + _NT - 1) // _NT


def _row_chunks(s, fn):
    def body(jj, _):
        ch = s + _NT * jj

        @pl.when(ch < _NRCH)
        def _():
            fn(ch * _C)
        return 0

    lax.fori_loop(0, _NJJ, body, 0)


def _zero_accum(zbuf, accum, s):
    _row_chunks(s, lambda r0: pltpu.sync_copy(zbuf, accum.at[pl.ds(r0, _C)]))


@functools.partial(
    pl.kernel,
    mesh=_mesh,
    out_type=[jax.ShapeDtypeStruct((_N, _CW), _f32),
              jax.ShapeDtypeStruct((_N, _CW), _f32)],
    scratch_types=[
        pltpu.VMEM((_C,), _i32),
        pltpu.VMEM((_C, _CW), _f32),
        pltpu.VMEM((_C, _CW), _f32),
        pltpu.VMEM_SHARED((_N, _CW), _f32),
    ],
    compiler_params=_sc_params,
)
def _sc_counts(src_hbm, dst_hbm, cnt_i_hbm, cnt_u_hbm, idxb, ones, zbuf, accum):
    c = lax.axis_index("c")
    s = lax.axis_index("s")
    _fill_const(ones, _C, _CW, 1.0)
    _fill_const(zbuf, _C, _CW, 0.0)
    _zero_accum(zbuf, accum, s)
    plsc.subcore_barrier()

    def body(j, _):
        e0 = s * _EPT + j * _C

        @pl.when(c == 0)
        def _():
            pltpu.sync_copy(dst_hbm.at[pl.ds(e0, _C)], idxb)

        @pl.when(c == 1)
        def _():
            pltpu.sync_copy(src_hbm.at[pl.ds(e0, _C)], idxb)

        pltpu.sync_copy(ones, accum.at[idxb], add=True)
        return 0

    lax.fori_loop(0, _NCH, body, 0)
    plsc.subcore_barrier()

    @pl.when(c == 0)
    def _():
        _row_chunks(s, lambda r0: pltpu.sync_copy(
            accum.at[pl.ds(r0, _C)], cnt_i_hbm.at[pl.ds(r0, _C)]))

    @pl.when(c == 1)
    def _():
        _row_chunks(s, lambda r0: pltpu.sync_copy(
            accum.at[pl.ds(r0, _C)], cnt_u_hbm.at[pl.ds(r0, _C)]))


def _make_segdir(grow):
    @functools.partial(
        pl.kernel,
        mesh=_mesh,
        out_type=[jax.ShapeDtypeStruct((_N, _H), _f32) for _ in range(2)],
        scratch_types=[
            pltpu.VMEM((2, _C), _i32),
            pltpu.VMEM((2, _C), _i32),
            pltpu.VMEM((_C, _H), _f32),
            pltpu.VMEM((_C, _H), _f32),
            pltpu.VMEM_SHARED((_N, _H), _f32),
            pltpu.SemaphoreType.DMA,
            pltpu.SemaphoreType.DMA,
        ],
        compiler_params=_sc_params,
    )
    def segdir(tbl0, tbl1, ed2_hbm, out0, out1,
               idx0, idx1, rows0, rows1, accum, sem0, sem1):
        c = lax.axis_index("c")
        s = lax.axis_index("s")
        idx = (idx0, idx1)
        rows = (rows0, rows1)
        sems = (sem0, sem1)

        _fill_const(rows0, _C, _H, 0.0)
        _zero_accum(rows0, accum, s)
        plsc.subcore_barrier()

        def issue(b, j):
            e0 = s * _EPT + j * _C
            pltpu.sync_copy(ed2_hbm.at[:, pl.ds(e0, _C)], idx[b])

            @pl.when(c == 0)
            def _():
                pltpu.async_copy(tbl0.at[idx[b].at[grow]], rows[b], sems[b])

            @pl.when(c == 1)
            def _():
                pltpu.async_copy(tbl1.at[idx[b].at[grow]], rows[b], sems[b])

        def drain(b):
            pltpu.make_async_copy(tbl0.at[pl.ds(0, _C)], rows[b],
                                  sems[b]).wait()
            pltpu.sync_copy(rows[b], accum.at[idx[b].at[1 - grow]], add=True)

        issue(0, 0)
        issue(1, 1)

        def body(jj, _):
            for b in range(2):
                drain(b)
                jn = 2 * jj + b + 2

                @pl.when(jn < _NCH)
                def _():
                    issue(b, jn)
            return 0

        lax.fori_loop(0, _NCH // 2, body, 0)
        drain(0)
        plsc.subcore_barrier()

        @pl.when(c == 0)
        def _():
            _row_chunks(s, lambda r0: pltpu.sync_copy(
                accum.at[pl.ds(r0, _C)], out0.at[pl.ds(r0, _C)]))

        @pl.when(c == 1)
        def _():
            _row_chunks(s, lambda r0: pltpu.sync_copy(
                accum.at[pl.ds(r0, _C)], out1.at[pl.ds(r0, _C)]))

    return segdir


_segdir_by_src = _make_segdir(0)
_segdir_by_dst = _make_segdir(1)


_GCH = _N // _C
_GJJ = (_GCH + _NT - 1) // _NT


@functools.partial(
    pl.kernel,
    mesh=_mesh,
    out_type=[jax.ShapeDtypeStruct((_N, _H), _f32) for _ in range(4)],
    scratch_types=[
        pltpu.VMEM((_C,), _i32),
        pltpu.VMEM((_C, _H), _f32),
        pltpu.SemaphoreType.DMA,
    ],
    compiler_params=_sc_params,
)
def _sc_gather_pairs(v0_hbm, v1_hbm, pos_hbm, neg_hbm,
                     pv0_hbm, pv1_hbm, nv0_hbm, nv1_hbm, idxb, rows, sem):
    c = lax.axis_index("c")
    s = lax.axis_index("s")

    def body(jj, _):
        ch = s + _NT * jj

        @pl.when(ch < _GCH)
        def _():
            b0 = ch * _C
            pltpu.sync_copy(pos_hbm.at[pl.ds(b0, _C)], idxb)

            @pl.when(c == 0)
            def _():
                pltpu.async_copy(v0_hbm.at[idxb], rows, sem).wait()
                pltpu.sync_copy(rows, pv0_hbm.at[pl.ds(b0, _C)])

            @pl.when(c == 1)
            def _():
                pltpu.async_copy(v1_hbm.at[idxb], rows, sem).wait()
                pltpu.sync_copy(rows, pv1_hbm.at[pl.ds(b0, _C)])

            pltpu.sync_copy(neg_hbm.at[pl.ds(b0, _C)], idxb)

            @pl.when(c == 0)
            def _():
                pltpu.async_copy(v0_hbm.at[idxb], rows, sem).wait()
                pltpu.sync_copy(rows, nv0_hbm.at[pl.ds(b0, _C)])

            @pl.when(c == 1)
            def _():
                pltpu.async_copy(v1_hbm.at[idxb], rows, sem).wait()
                pltpu.sync_copy(rows, nv1_hbm.at[pl.ds(b0, _C)])

        return 0

    lax.fori_loop(0, _GJJ, body, 0)


_R = 2000


def _dgT(x, w):
    return lax.dot_general(x, w, (((1,), (1,)), ((), ())),
                           preferred_element_type=_f32)


def _tc_half_body(relu, x0r, x1r, s0r, s1r, cn, ws, wn, o0, o1):
    cnc = cn[:, 0:1]
    wsv = ws[:, :]
    wnv = wn[:, :]

    def mean(s):
        return jnp.where(cnc > 0, s / jnp.maximum(cnc, 1.0), 0.0)

    m0 = mean(s0r[:, :])
    m1 = mean(s1r[:, :])
    a0, a1 = x0r[:, :], x1r[:, :]
    r0 = (_dgT(a0, wsv[0:32, 0:32]) + _dgT(a1, wsv[0:32, 32:64])
          + _dgT(m0, wnv[0:32, 0:32]) + _dgT(m1, wnv[0:32, 32:64]))
    r1 = (_dgT(a0, wsv[32:64, 0:32]) + _dgT(a1, wsv[32:64, 32:64])
          + _dgT(m0, wnv[32:64, 0:32]) + _dgT(m1, wnv[32:64, 32:64]))
    if relu:
        r0 = jnp.maximum(r0, 0.0)
        r1 = jnp.maximum(r1, 0.0)
    o0[:, :] = r0
    o1[:, :] = r1


def _tc_half(relu):
    half = pl.BlockSpec((_R, _H), lambda i: (i, 0))
    cnts = pl.BlockSpec((_R, _CW), lambda i: (i, 0))
    wspec = pl.BlockSpec((_D, _D), lambda i: (0, 0))
    return pl.pallas_call(
        functools.partial(_tc_half_body, relu),
        grid=(_N // _R,),
        in_specs=[half] * 4 + [cnts] + [wspec] * 2,
        out_specs=[half] * 2,
        out_shape=[jax.ShapeDtypeStruct((_N, _H), _f32) for _ in range(2)],
    )


def _tc_loss_body(u0, u1, pv0, pv1, nv0, nv1, mf, sq):
    i = pl.program_id(0)
    x0, x1 = u0[:, :], u1[:, :]
    p0, p1 = pv0[:, :], pv1[:, :]
    n0, n1 = nv0[:, :], nv1[:, :]
    ps = jnp.sum(x0 * p0 + x1 * p1, axis=1)
    ns = jnp.sum(x0 * n0 + x1 * n1, axis=1)
    x = ns - ps
    sp = jnp.maximum(x, 0.0) + jnp.log1p(jnp.exp(-jnp.abs(x)))
    blk_mf = jnp.sum(sp)
    blk_sq = (jnp.sum(x0 * x0) + jnp.sum(x1 * x1)
              + jnp.sum(p0 * p0) + jnp.sum(p1 * p1)
              + jnp.sum(n0 * n0) + jnp.sum(n1 * n1))

    @pl.when(i == 0)
    def _():
        mf[0, 0] = blk_mf
        sq[0, 0] = blk_sq

    @pl.when(i > 0)
    def _():
        mf[0, 0] += blk_mf
        sq[0, 0] += blk_sq


_tc_loss = pl.pallas_call(
    _tc_loss_body,
    grid=(_N // _R,),
    in_specs=[pl.BlockSpec((_R, _H), lambda i: (i, 0))] * 6,
    out_specs=[pl.BlockSpec(memory_space=pltpu.MemorySpace.SMEM)] * 2,
    out_shape=[jax.ShapeDtypeStruct((1, 1), _f32) for _ in range(2)],
)


def kernel(user_feat, item_feat, edge_index, pos_items_ur, neg_items_ur,
           W_self_0, W_neigh_0, W_self_1, W_neigh_1):
    src = edge_index[0].astype(_i32)
    dst = edge_index[1].astype(_i32)
    pos = pos_items_ur.astype(_i32)
    neg = neg_items_ur.astype(_i32)

    u0, u1 = user_feat[:, :_H], user_feat[:, _H:]
    v0, v1 = item_feat[:, :_H], item_feat[:, _H:]

    ed2 = jnp.stack([src, dst])
    cnt_i, cnt_u = _sc_counts(src, dst)

    si0, si1 = _segdir_by_src(u0, u1, ed2)
    su0, su1 = _segdir_by_dst(v0, v1, ed2)
    v10, v11 = _tc_half(True)(v0, v1, si0, si1, cnt_i, W_self_0, W_neigh_0)
    su20, su21 = _segdir_by_dst(v10, v11, ed2)
    u10, u11 = _tc_half(True)(u0, u1, su0, su1, cnt_u, W_self_0, W_neigh_0)
    si20, si21 = _segdir_by_src(u10, u11, ed2)
    u20, u21 = _tc_half(False)(u10, u11, su20, su21, cnt_u,
                               W_self_1, W_neigh_1)
    v20, v21 = _tc_half(False)(v10, v11, si20, si21, cnt_i,
                               W_self_1, W_neigh_1)

    pv0, pv1, nv0, nv1 = _sc_gather_pairs(v20, v21, pos, neg)
    mf, sq = _tc_loss(u20, u21, pv0, pv1, nv0, nv1)

    mf_loss = (mf[0, 0] / _BATCH).astype(_f32)
    emb_loss = (_DECAY * 0.5 * sq[0, 0] / _BATCH).astype(_f32)
    return (mf_loss, emb_loss)


# speedup vs baseline: 1.7881x; 1.2988x over previous
import functools

import jax
import jax.numpy as jnp
from jax import lax
from jax.experimental import pallas as pl
from jax.experimental.pallas import tpu as pltpu
from jax.experimental.pallas import tpu_sc as plsc

_N = 50000
_D = 64
_H = 32
_E = 800000
_BATCH = 50000
_DECAY = 1e-05

_NT = 16
_EPT = _E // _NT
_C = 400
_NCH = _EPT // _C
_CW = 16

_mesh = plsc.VectorSubcoreMesh(core_axis_name="c", subcore_axis_name="s")
_sc_params = pltpu.CompilerParams(use_tc_tiling_on_sc=False)
_f32 = jnp.float32
_i32 = jnp.int32


def _fill_const(ref, nrows, width, val):
    def body(r, _):
        for h in range(width // 16):
            ref[r, pl.ds(h * 16, 16)] = jnp.full((16,), val, _f32)
        return 0
    lax.fori_loop(0, nrows, body, 0)


_NRCH = _N // _C
_NJJ = (_NRCH + _NT - 1) // _NT


def _row_chunks(s, fn):
    def body(jj, _):
        ch = s + _NT * jj

        @pl.when(ch < _NRCH)
        def _():
            fn(ch * _C)
        return 0

    lax.fori_loop(0, _NJJ, body, 0)


def _zero_accum(zbuf, accum, s):
    _row_chunks(s, lambda r0: pltpu.sync_copy(zbuf, accum.at[pl.ds(r0, _C)]))


@functools.partial(
    pl.kernel,
    mesh=_mesh,
    out_type=[jax.ShapeDtypeStruct((_N, _CW), _f32),
              jax.ShapeDtypeStruct((_N, _CW), _f32)],
    scratch_types=[
        pltpu.VMEM((_C,), _i32),
        pltpu.VMEM((_C, _CW), _f32),
        pltpu.VMEM((_C, _CW), _f32),
        pltpu.VMEM_SHARED((_N, _CW), _f32),
    ],
    compiler_params=_sc_params,
)
def _sc_counts(src_hbm, dst_hbm, cnt_i_hbm, cnt_u_hbm, idxb, ones, zbuf, accum):
    c = lax.axis_index("c")
    s = lax.axis_index("s")
    _fill_const(ones, _C, _CW, 1.0)
    _fill_const(zbuf, _C, _CW, 0.0)
    _zero_accum(zbuf, accum, s)
    plsc.subcore_barrier()

    def body(j, _):
        e0 = s * _EPT + j * _C

        @pl.when(c == 0)
        def _():
            pltpu.sync_copy(dst_hbm.at[pl.ds(e0, _C)], idxb)

        @pl.when(c == 1)
        def _():
            pltpu.sync_copy(src_hbm.at[pl.ds(e0, _C)], idxb)

        pltpu.sync_copy(ones, accum.at[idxb], add=True)
        return 0

    lax.fori_loop(0, _NCH, body, 0)
    plsc.subcore_barrier()

    @pl.when(c == 0)
    def _():
        _row_chunks(s, lambda r0: pltpu.sync_copy(
            accum.at[pl.ds(r0, _C)], cnt_i_hbm.at[pl.ds(r0, _C)]))

    @pl.when(c == 1)
    def _():
        _row_chunks(s, lambda r0: pltpu.sync_copy(
            accum.at[pl.ds(r0, _C)], cnt_u_hbm.at[pl.ds(r0, _C)]))


def _make_segdir(grow):
    @functools.partial(
        pl.kernel,
        mesh=_mesh,
        out_type=[jax.ShapeDtypeStruct((_N, _H), _f32) for _ in range(2)],
        scratch_types=[
            pltpu.VMEM((2, _C), _i32),
            pltpu.VMEM((2, _C), _i32),
            pltpu.VMEM((_C, _H), _f32),
            pltpu.VMEM((_C, _H), _f32),
            pltpu.VMEM_SHARED((_N, _H), _f32),
            pltpu.SemaphoreType.DMA,
            pltpu.SemaphoreType.DMA,
        ],
        compiler_params=_sc_params,
    )
    def segdir(tbl0, tbl1, ed2_hbm, out0, out1,
               idx0, idx1, rows0, rows1, accum, sem0, sem1):
        c = lax.axis_index("c")
        s = lax.axis_index("s")
        idx = (idx0, idx1)
        rows = (rows0, rows1)
        sems = (sem0, sem1)

        _fill_const(rows0, _C, _H, 0.0)
        _zero_accum(rows0, accum, s)
        plsc.subcore_barrier()

        def issue(b, j):
            e0 = s * _EPT + j * _C
            pltpu.sync_copy(ed2_hbm.at[:, pl.ds(e0, _C)], idx[b])

            @pl.when(c == 0)
            def _():
                pltpu.async_copy(tbl0.at[idx[b].at[grow]], rows[b], sems[b])

            @pl.when(c == 1)
            def _():
                pltpu.async_copy(tbl1.at[idx[b].at[grow]], rows[b], sems[b])

        def drain(b):
            pltpu.make_async_copy(tbl0.at[pl.ds(0, _C)], rows[b],
                                  sems[b]).wait()
            pltpu.sync_copy(rows[b], accum.at[idx[b].at[1 - grow]], add=True)

        issue(0, 0)
        issue(1, 1)

        def body(jj, _):
            for b in range(2):
                drain(b)
                jn = 2 * jj + b + 2

                @pl.when(jn < _NCH)
                def _():
                    issue(b, jn)
            return 0

        lax.fori_loop(0, _NCH // 2, body, 0)
        drain(0)
        plsc.subcore_barrier()

        @pl.when(c == 0)
        def _():
            _row_chunks(s, lambda r0: pltpu.sync_copy(
                accum.at[pl.ds(r0, _C)], out0.at[pl.ds(r0, _C)]))

        @pl.when(c == 1)
        def _():
            _row_chunks(s, lambda r0: pltpu.sync_copy(
                accum.at[pl.ds(r0, _C)], out1.at[pl.ds(r0, _C)]))

    return segdir


_segdir_by_src = _make_segdir(0)
_segdir_by_dst = _make_segdir(1)


_GCH = _N // _C
_GJJ = (_GCH + _NT - 1) // _NT


@functools.partial(
    pl.kernel,
    mesh=_mesh,
    out_type=[jax.ShapeDtypeStruct((_N, _H), _f32) for _ in range(4)],
    scratch_types=[
        pltpu.VMEM((_C,), _i32),
        pltpu.VMEM((_C, _H), _f32),
        pltpu.SemaphoreType.DMA,
    ],
    compiler_params=_sc_params,
)
def _sc_gather_pairs(v0_hbm, v1_hbm, pos_hbm, neg_hbm,
                     pv0_hbm, pv1_hbm, nv0_hbm, nv1_hbm, idxb, rows, sem):
    c = lax.axis_index("c")
    s = lax.axis_index("s")

    def body(jj, _):
        ch = s + _NT * jj

        @pl.when(ch < _GCH)
        def _():
            b0 = ch * _C
            pltpu.sync_copy(pos_hbm.at[pl.ds(b0, _C)], idxb)

            @pl.when(c == 0)
            def _():
                pltpu.async_copy(v0_hbm.at[idxb], rows, sem).wait()
                pltpu.sync_copy(rows, pv0_hbm.at[pl.ds(b0, _C)])

            @pl.when(c == 1)
            def _():
                pltpu.async_copy(v1_hbm.at[idxb], rows, sem).wait()
                pltpu.sync_copy(rows, pv1_hbm.at[pl.ds(b0, _C)])

            pltpu.sync_copy(neg_hbm.at[pl.ds(b0, _C)], idxb)

            @pl.when(c == 0)
            def _():
                pltpu.async_copy(v0_hbm.at[idxb], rows, sem).wait()
                pltpu.sync_copy(rows, nv0_hbm.at[pl.ds(b0, _C)])

            @pl.when(c == 1)
            def _():
                pltpu.async_copy(v1_hbm.at[idxb], rows, sem).wait()
                pltpu.sync_copy(rows, nv1_hbm.at[pl.ds(b0, _C)])

        return 0

    lax.fori_loop(0, _GJJ, body, 0)


_R = 2000


def _dgT(x, w):
    return lax.dot_general(x, w, (((1,), (1,)), ((), ())),
                           preferred_element_type=_f32)


def _tc_half_body(relu, x0r, x1r, s0r, s1r, rb, kw, o0, o1):
    rbv = rb[:, :]
    w = kw[:, :]
    a0 = x0r[:, :]
    a1 = x1r[:, :]
    m0 = s0r[:, :] * rbv
    m1 = s1r[:, :] * rbv

    def dot(x, k):
        return jnp.dot(x, w[128 * k:128 * (k + 1), :],
                       preferred_element_type=_f32)

    r0 = dot(a0, 0) + dot(a1, 1) + dot(m0, 2) + dot(m1, 3)
    r1 = dot(a0, 4) + dot(a1, 5) + dot(m0, 6) + dot(m1, 7)
    if relu:
        r0 = jnp.maximum(r0, 0.0)
        r1 = jnp.maximum(r1, 0.0)
    o0[:, :] = r0
    o1[:, :] = r1


_PK = pl.BlockSpec((_N // 4, 4 * _H), lambda: (0, 0))


def _pk(a):
    return jnp.reshape(a, (_N // 4, 4 * _H))


def _tc_half(relu):
    wspec = pl.BlockSpec((8 * 4 * _H, 4 * _H), lambda: (0, 0))
    return pl.pallas_call(
        functools.partial(_tc_half_body, relu),
        in_specs=[_PK] * 5 + [wspec],
        out_specs=[_PK] * 2,
        out_shape=[jax.ShapeDtypeStruct((_N // 4, 4 * _H), _f32)
                   for _ in range(2)],
    )


def _kron_w(ws, wn):
    eye = jnp.eye(4, dtype=_f32)

    def q(wm, a, b):
        return jnp.kron(eye, wm[32 * a:32 * a + 32, 32 * b:32 * b + 32].T)

    return jnp.concatenate(
        [q(ws, 0, 0), q(ws, 0, 1), q(wn, 0, 0), q(wn, 0, 1),
         q(ws, 1, 0), q(ws, 1, 1), q(wn, 1, 0), q(wn, 1, 1)], axis=0)


def _tc_loss_body(u0, u1, pv0, pv1, nv0, nv1, ob, mf, sq):
    x0, x1 = u0[:, :], u1[:, :]
    p0, p1 = pv0[:, :], pv1[:, :]
    n0, n1 = nv0[:, :], nv1[:, :]
    d = jnp.dot(x0 * (n0 - p0) + x1 * (n1 - p1), ob[:, :],
                preferred_element_type=_f32)
    sp = jnp.maximum(d, 0.0) + jnp.log1p(jnp.exp(-jnp.abs(d)))
    mf[0, 0] = jnp.sum(sp) * (1.0 / 32.0)
    sq[0, 0] = (jnp.sum(x0 * x0) + jnp.sum(x1 * x1)
                + jnp.sum(p0 * p0) + jnp.sum(p1 * p1)
                + jnp.sum(n0 * n0) + jnp.sum(n1 * n1))


_tc_loss = pl.pallas_call(
    _tc_loss_body,
    in_specs=[_PK] * 6 + [pl.BlockSpec((4 * _H, 4 * _H), lambda: (0, 0))],
    out_specs=[pl.BlockSpec(memory_space=pltpu.MemorySpace.SMEM)] * 2,
    out_shape=[jax.ShapeDtypeStruct((1, 1), _f32) for _ in range(2)],
)


def kernel(user_feat, item_feat, edge_index, pos_items_ur, neg_items_ur,
           W_self_0, W_neigh_0, W_self_1, W_neigh_1):
    src = edge_index[0].astype(_i32)
    dst = edge_index[1].astype(_i32)
    pos = pos_items_ur.astype(_i32)
    neg = neg_items_ur.astype(_i32)

    u0, u1 = user_feat[:, :_H], user_feat[:, _H:]
    v0, v1 = item_feat[:, :_H], item_feat[:, _H:]

    ed2 = jnp.stack([src, dst])
    cnt_i, cnt_u = _sc_counts(src, dst)

    def unpk(a):
        return jnp.reshape(a, (_N, _H))

    def recip_pk(cnt):
        r = 1.0 / jnp.maximum(cnt[:, 0:1], 1.0)
        return _pk(jnp.broadcast_to(r, (_N, _H)))

    ci_p = recip_pk(cnt_i)
    cu_p = recip_pk(cnt_u)

    si0, si1 = _segdir_by_src(u0, u1, ed2)
    su0, su1 = _segdir_by_dst(v0, v1, ed2)
    kw0 = _kron_w(W_self_0, W_neigh_0)
    kw1 = _kron_w(W_self_1, W_neigh_1)
    ob = jnp.kron(jnp.eye(4, dtype=_f32), jnp.ones((_H, _H), _f32))

    v1p0, v1p1 = _tc_half(True)(_pk(v0), _pk(v1), _pk(si0), _pk(si1),
                                ci_p, kw0)
    su20, su21 = _segdir_by_dst(unpk(v1p0), unpk(v1p1), ed2)
    u1p0, u1p1 = _tc_half(True)(_pk(u0), _pk(u1), _pk(su0), _pk(su1),
                                cu_p, kw0)
    si20, si21 = _segdir_by_src(unpk(u1p0), unpk(u1p1), ed2)
    u2p0, u2p1 = _tc_half(False)(u1p0, u1p1, _pk(su20), _pk(su21),
                                 cu_p, kw1)
    v2p0, v2p1 = _tc_half(False)(v1p0, v1p1, _pk(si20), _pk(si21),
                                 ci_p, kw1)

    pv0, pv1, nv0, nv1 = _sc_gather_pairs(unpk(v2p0), unpk(v2p1), pos, neg)
    mf, sq = _tc_loss(u2p0, u2p1, _pk(pv0), _pk(pv1), _pk(nv0), _pk(nv1), ob)

    mf_loss = (mf[0, 0] / _BATCH).astype(_f32)
    emb_loss = (_DECAY * 0.5 * sq[0, 0] / _BATCH).astype(_f32)
    return (mf_loss, emb_loss)
